# Initial kernel scaffold; baseline (speedup 1.0000x reference)
import jax
import jax.numpy as jnp
from jax.experimental import pallas as pl


def kernel(x, embeddings):
    raise NotImplementedError("write your pallas kernel here")


# speedup vs baseline: 131.5725x; 131.5725x over previous
import jax
import jax.numpy as jnp
from jax.experimental import pallas as pl


def _band_mean_kernel(emb_ref, out_ref):
    S = out_ref.shape[1]
    R = emb_ref.shape[0]
    E = emb_ref[...]
    i = jax.lax.broadcasted_iota(jnp.int32, (S, R), 0)
    k = jax.lax.broadcasted_iota(jnp.int32, (S, R), 1)
    band = ((k >= i) & (k - i < S)).astype(jnp.float32)
    out_ref[0] = jnp.dot(band, E, preferred_element_type=jnp.float32) * (1.0 / S)


def kernel(x, embeddings):
    B, S, H = x.shape
    emb = jnp.pad(embeddings, ((0, 1), (0, 0)))
    return pl.pallas_call(
        _band_mean_kernel,
        grid=(B,),
        in_specs=[pl.BlockSpec((2 * S, H), lambda b: (0, 0))],
        out_specs=pl.BlockSpec((1, S, H), lambda b: (b, 0, 0)),
        out_shape=jax.ShapeDtypeStruct((B, S, H), jnp.float32),
    )(emb)
